# baseline (speedup 1.0000x reference)
import functools

import jax
import jax.numpy as jnp
from jax import lax
from jax.experimental import pallas as pl
from jax.experimental.pallas import tpu as pltpu
from jax.experimental.pallas import tpu_sc as plsc


@functools.cache
def _make_gather(V, D, B, S):
    info = plsc.get_sparse_core_info()
    NC, NS = info.num_cores, info.num_subcores
    NW = NC * NS
    N = B * S
    assert N % NW == 0
    n_per_w = N // NW
    assert n_per_w % 8 == 0 and S % n_per_w == 0
    wpb = S // n_per_w
    mesh = plsc.VectorSubcoreMesh(core_axis_name="c", subcore_axis_name="s")

    NCH = 4
    assert n_per_w % NCH == 0
    C = n_per_w // NCH
    assert C % 8 == 0

    @functools.partial(
        pl.kernel,
        mesh=mesh,
        out_type=jax.ShapeDtypeStruct((B, S, D), jnp.float32),
        scratch_types=[
            pltpu.VMEM((n_per_w,), jnp.int32),
            pltpu.VMEM((C, D), jnp.float32),
            pltpu.VMEM((C, D), jnp.float32),
            pltpu.SemaphoreType.DMA,
            pltpu.SemaphoreType.DMA,
            pltpu.SemaphoreType.DMA,
            pltpu.SemaphoreType.DMA,
        ],
    )
    def gather_kernel(table_hbm, idx_hbm, out_hbm, idx_v, buf_a, buf_b,
                      gs_a, gs_b, ss_a, ss_b):
        wid = lax.axis_index("s") * NC + lax.axis_index("c")
        b = wid // wpb
        s0 = (wid % wpb) * n_per_w
        pltpu.sync_copy(idx_hbm.at[b, pl.ds(s0, n_per_w)], idx_v)
        bufs, gs, ss = (buf_a, buf_b), (gs_a, gs_b), (ss_a, ss_b)

        def start_g(i):
            return pltpu.async_copy(
                table_hbm.at[idx_v.at[pl.ds(i * C, C)]], bufs[i % 2], gs[i % 2])

        def start_s(i):
            return pltpu.async_copy(
                bufs[i % 2], out_hbm.at[b, pl.ds(s0 + i * C, C)], ss[i % 2])

        g = [None] * NCH
        s = [None] * NCH
        g[0] = start_g(0)
        for i in range(NCH):
            g[i].wait()
            if i + 1 < NCH:
                if i >= 1:
                    s[i - 1].wait()
                g[i + 1] = start_g(i + 1)
            s[i] = start_s(i)
        s[NCH - 2].wait()
        s[NCH - 1].wait()

    return gather_kernel


def kernel(input_ids, embed_weight):
    B, S = input_ids.shape
    V, D = embed_weight.shape
    return _make_gather(V, D, B, S)(embed_weight, input_ids)


# speedup vs baseline: 1.0753x; 1.0753x over previous
import functools

import jax
import jax.numpy as jnp
from jax import lax
from jax.experimental import pallas as pl
from jax.experimental.pallas import tpu as pltpu
from jax.experimental.pallas import tpu_sc as plsc


@functools.cache
def _make_gather(V, D, B, S):
    info = plsc.get_sparse_core_info()
    NC, NS = info.num_cores, info.num_subcores
    NW = NC * NS
    N = B * S
    assert N % NW == 0
    n_per_w = N // NW
    assert n_per_w % 8 == 0 and S % n_per_w == 0
    wpb = S // n_per_w
    mesh = plsc.VectorSubcoreMesh(core_axis_name="c", subcore_axis_name="s")

    @functools.partial(
        pl.kernel,
        mesh=mesh,
        out_type=jax.ShapeDtypeStruct((B, S, D), jnp.float32),
        scratch_types=[
            pltpu.VMEM((n_per_w,), jnp.int32),
            pltpu.VMEM((n_per_w, D), jnp.float32),
            pltpu.SemaphoreType.DMA,
        ],
    )
    def gather_kernel(table_hbm, idx_hbm, out_hbm, idx_v, rows_v, sem):
        wid = lax.axis_index("s") * NC + lax.axis_index("c")
        b = wid // wpb
        s0 = (wid % wpb) * n_per_w
        pltpu.sync_copy(idx_hbm.at[b, pl.ds(s0, n_per_w)], idx_v)
        pltpu.async_copy(table_hbm.at[idx_v], rows_v, sem).wait()
        pltpu.sync_copy(rows_v, out_hbm.at[b, pl.ds(s0, n_per_w)])

    return gather_kernel


def kernel(input_ids, embed_weight):
    B, S = input_ids.shape
    V, D = embed_weight.shape
    return _make_gather(V, D, B, S)(embed_weight, input_ids)
